# Initial kernel scaffold; baseline (speedup 1.0000x reference)
import jax
import jax.numpy as jnp
from jax.experimental import pallas as pl


def kernel(anchor_x, anchor_edge_index, anchor_batch, positive_x, positive_edge_index, positive_batch, negative_x, negative_edge_index, negative_batch, params):
    raise NotImplementedError("write your pallas kernel here")


# speedup vs baseline: 7.0148x; 7.0148x over previous
import functools

import jax
import jax.numpy as jnp
from jax import lax
from jax.experimental import pallas as pl
from jax.experimental.pallas import tpu as pltpu
from jax.experimental.pallas import tpu_sc as plsc

N = 10000
D = 128
H = 128
E = 320000
G = 16
OUT = 64

N_PAD = 10240
BLK = 1280
N_BLOCKS = N_PAD // BLK

NW = 32
CH = 79
E_PAD = NW * CH * 128
RPT = N_PAD // 16

DEG_CH = 237
DEG_PT = 3 * N_PAD // 16


@functools.cache
def _get_deg_call():
    mesh = plsc.VectorSubcoreMesh(core_axis_name="c", subcore_axis_name="s")

    @functools.partial(
        pl.kernel,
        mesh=mesh,
        out_type=jax.ShapeDtypeStruct((2, 3 * N_PAD), jnp.float32),
        scratch_types=[
            pltpu.VMEM((DEG_CH, 128), jnp.int32),
            pltpu.VMEM((128,), jnp.float32),
            pltpu.VMEM_SHARED((3 * N_PAD,), jnp.float32),
        ],
    )
    def deg_call(dst_hbm, ones_hbm, zeros_hbm, out_hbm, didx, ones_v, acc):
        cid = lax.axis_index("c")
        sid = lax.axis_index("s")
        w = cid * 16 + sid
        z0 = sid * DEG_PT
        pltpu.sync_copy(zeros_hbm.at[pl.ds(z0, DEG_PT)], acc.at[pl.ds(z0, DEG_PT)])
        pltpu.sync_copy(ones_hbm, ones_v)
        pltpu.sync_copy(dst_hbm.at[w], didx)
        plsc.subcore_barrier()

        def body(j, carry):
            pltpu.sync_copy(ones_v, acc.at[didx.at[j]], add=True)
            return carry

        lax.fori_loop(0, DEG_CH, body, 0)
        plsc.subcore_barrier()
        pltpu.sync_copy(acc.at[pl.ds(z0, DEG_PT)], out_hbm.at[cid].at[pl.ds(z0, DEG_PT)])

    return deg_call


def _deg_call(dst_hbm, ones_hbm, zeros_hbm):
    return _get_deg_call()(dst_hbm, ones_hbm, zeros_hbm)


@functools.cache
def _get_edge_agg():
    mesh = plsc.VectorSubcoreMesh(core_axis_name="c", subcore_axis_name="s")

    @functools.partial(
        pl.kernel,
        mesh=mesh,
        out_type=jax.ShapeDtypeStruct((2, N_PAD, 128), jnp.float32),
        scratch_types=[
            pltpu.VMEM((CH, 128), jnp.int32),
            pltpu.VMEM((CH, 128), jnp.int32),
            pltpu.VMEM((128, 128), jnp.float32),
            pltpu.VMEM_SHARED((N_PAD, 128), jnp.float32),
            pltpu.SemaphoreType.DMA,
        ],
    )
    def edge_agg(h_hbm, src_hbm, dst_hbm, zeros_hbm, out_hbm,
                 sidx, didx, rows, acc, sem):
        cid = lax.axis_index("c")
        sid = lax.axis_index("s")
        w = cid * 16 + sid
        r0 = sid * RPT

        @pl.when(cid == 0)
        def _():
            pltpu.sync_copy(h_hbm.at[pl.ds(r0, RPT)], acc.at[pl.ds(r0, RPT)])

        @pl.when(cid != 0)
        def _():
            pltpu.sync_copy(zeros_hbm.at[pl.ds(r0, RPT)], acc.at[pl.ds(r0, RPT)])

        pltpu.sync_copy(src_hbm.at[w], sidx)
        pltpu.sync_copy(dst_hbm.at[w], didx)
        plsc.subcore_barrier()

        def body(j, carry):
            pltpu.async_copy(h_hbm.at[sidx.at[j]], rows, sem).wait()
            pltpu.sync_copy(rows, acc.at[didx.at[j]], add=True)
            return carry

        lax.fori_loop(0, CH, body, 0)
        plsc.subcore_barrier()
        pltpu.sync_copy(acc.at[pl.ds(r0, RPT)], out_hbm.at[cid].at[pl.ds(r0, RPT)])

    return edge_agg


def _edge_agg(h_hbm, src_hbm, dst_hbm, zeros_hbm):
    return _get_edge_agg()(h_hbm, src_hbm, dst_hbm, zeros_hbm)


def _dis(i, deg0_ref, deg1_ref):
    deg = deg0_ref[...] + deg1_ref[...] + 1.0
    rows = i * BLK + lax.broadcasted_iota(jnp.int32, (BLK, 1), 0)
    return jnp.where(rows < N, lax.rsqrt(deg), 0.0)


def _m1_body(x_ref, w_ref, deg0_ref, deg1_ref, o_ref):
    i = pl.program_id(0)
    dis = _dis(i, deg0_ref, deg1_ref)
    o_ref[...] = dis * jnp.dot(x_ref[...], w_ref[...],
                               preferred_element_type=jnp.float32)


def _m2_body(p0_ref, p1_ref, deg0_ref, deg1_ref, b_ref, w_ref, o_ref):
    i = pl.program_id(0)
    dis = _dis(i, deg0_ref, deg1_ref)
    x = jax.nn.relu(dis * (p0_ref[...] + p1_ref[...]) + b_ref[...])
    o_ref[...] = dis * jnp.dot(x, w_ref[...], preferred_element_type=jnp.float32)


def _m3_body(p0_ref, p1_ref, deg0_ref, deg1_ref, b_ref, batch_ref,
             fw1_ref, fb1_ref, fw2_ref, fb2_ref, fw3_ref, fb3_ref,
             o_ref, pooled_acc, cnt_acc):
    i = pl.program_id(0)

    @pl.when(i == 0)
    def _():
        pooled_acc[...] = jnp.zeros((128, 128), jnp.float32)
        cnt_acc[...] = jnp.zeros((128, 128), jnp.float32)

    dis = _dis(i, deg0_ref, deg1_ref)
    x5 = dis * (p0_ref[...] + p1_ref[...]) + b_ref[...]
    g_iota = lax.broadcasted_iota(jnp.int32, (BLK, 128), 1)
    mask = (batch_ref[...] == g_iota).astype(jnp.float32)
    dn = (((0,), (0,)), ((), ()))
    pooled_acc[...] += lax.dot_general(mask, x5, dn,
                                       preferred_element_type=jnp.float32)
    cnt_acc[...] += lax.dot_general(mask, jnp.ones((BLK, 128), jnp.float32), dn,
                                    preferred_element_type=jnp.float32)

    @pl.when(i == N_BLOCKS - 1)
    def _():
        mean = pooled_acc[...] / jnp.maximum(cnt_acc[...], 1.0)
        h1 = jax.nn.relu(jnp.dot(mean, fw1_ref[...],
                                 preferred_element_type=jnp.float32) + fb1_ref[...])
        h2 = jax.nn.relu(jnp.dot(h1, fw2_ref[...],
                                 preferred_element_type=jnp.float32) + fb2_ref[...])
        o_ref[...] = jnp.dot(h2, fw3_ref[...],
                             preferred_element_type=jnp.float32) + fb3_ref[...]


def _row_spec():
    return pl.BlockSpec((BLK, 128), lambda i: (i, 0))


def _col_spec():
    return pl.BlockSpec((BLK, 1), lambda i: (i, 0))


def _fix_spec(shape):
    return pl.BlockSpec(shape, lambda i: tuple(0 for _ in shape))


_m1 = pl.pallas_call(
    _m1_body,
    grid=(N_BLOCKS,),
    in_specs=[_row_spec(), _fix_spec((128, 128)), _col_spec(), _col_spec()],
    out_specs=_row_spec(),
    out_shape=jax.ShapeDtypeStruct((N_PAD, 128), jnp.float32),
)

_m2 = pl.pallas_call(
    _m2_body,
    grid=(N_BLOCKS,),
    in_specs=[_row_spec(), _row_spec(), _col_spec(), _col_spec(),
              _fix_spec((1, 128)), _fix_spec((128, 128))],
    out_specs=_row_spec(),
    out_shape=jax.ShapeDtypeStruct((N_PAD, 128), jnp.float32),
)

_m3 = pl.pallas_call(
    _m3_body,
    grid=(N_BLOCKS,),
    in_specs=[_row_spec(), _row_spec(), _col_spec(), _col_spec(),
              _fix_spec((1, 128)), _col_spec(),
              _fix_spec((128, 128)), _fix_spec((1, 128)),
              _fix_spec((128, 128)), _fix_spec((1, 128)),
              _fix_spec((128, OUT)), _fix_spec((1, OUT))],
    out_specs=_fix_spec((128, OUT)),
    out_shape=jax.ShapeDtypeStruct((128, OUT), jnp.float32),
    scratch_shapes=[pltpu.VMEM((128, 128), jnp.float32),
                    pltpu.VMEM((128, 128), jnp.float32)],
)


def kernel(anchor_x, anchor_edge_index, anchor_batch,
           positive_x, positive_edge_index, positive_batch,
           negative_x, negative_edge_index, negative_batch, params):
    f32 = jnp.float32
    i32 = jnp.int32

    def prep(x, ei, batch):
        x_pad = jnp.pad(x, ((0, N_PAD - N), (0, 0)))
        src = jnp.pad(ei[0], (0, E_PAD - E), constant_values=N)
        dst = jnp.pad(ei[1], (0, E_PAD - E), constant_values=N)
        batch_pad = jnp.pad(batch, (0, N_PAD - N), constant_values=G)
        return x_pad, src, dst, batch_pad.astype(i32).reshape(N_PAD, 1)

    branches = [prep(anchor_x, anchor_edge_index, anchor_batch),
                prep(positive_x, positive_edge_index, positive_batch),
                prep(negative_x, negative_edge_index, negative_batch)]

    ones128 = jnp.ones((128,), f32)
    zeros3n = jnp.zeros((3 * N_PAD,), f32)
    zeros2d = jnp.zeros((N_PAD, 128), f32)

    deg_dst = jnp.concatenate([branches[b][2] + b * N_PAD for b in range(3)])
    deg_dst = deg_dst.reshape(NW, DEG_CH, 128)
    degp = _deg_call(deg_dst, ones128, zeros3n)

    Ws = [params[f"W{i}"] for i in range(1, 6)]
    bs = [params[f"b{i}"].reshape(1, H) for i in range(1, 6)]
    fws = [params[f"fcW{i}"] for i in range(1, 4)]
    fbs = [params["fcb1"].reshape(1, 128), params["fcb2"].reshape(1, 128),
           params["fcb3"].reshape(1, OUT)]

    outs = []
    for b, (x_pad, src, dst, batch_pad) in enumerate(branches):
        d0 = degp[0, b * N_PAD:(b + 1) * N_PAD].reshape(N_PAD, 1)
        d1 = degp[1, b * N_PAD:(b + 1) * N_PAD].reshape(N_PAD, 1)
        src3 = src.reshape(NW, CH, 128)
        dst3 = dst.reshape(NW, CH, 128)

        hp = _m1(x_pad, Ws[0], d0, d1)
        for l in range(4):
            parts = _edge_agg(hp, src3, dst3, zeros2d)
            hp = _m2(parts[0], parts[1], d0, d1, bs[l], Ws[l + 1])
        parts = _edge_agg(hp, src3, dst3, zeros2d)
        o = _m3(parts[0], parts[1], d0, d1, bs[4], batch_pad,
                fws[0], fbs[0], fws[1], fbs[1], fws[2], fbs[2])
        outs.append(o[:G, :])

    return tuple(outs)
